# baseline (speedup 1.0000x reference)
import jax
import jax.numpy as jnp
from jax.experimental import pallas as pl
from jax.experimental.pallas import tpu as pltpu

SEQ = 4096
FEAT = 26
DIM = 64
BATCH = 4

BS = 256


def _add_body(x_ref, t_ref, o_ref):
    o_ref[...] = x_ref[...] + t_ref[None]


def kernel(inputs, table):
    t3 = table.reshape(SEQ, FEAT, DIM)
    return pl.pallas_call(
        _add_body,
        grid=(SEQ // BS, BATCH),
        in_specs=[
            pl.BlockSpec((1, BS, FEAT, DIM), lambda s, b: (b, s, 0, 0)),
            pl.BlockSpec((BS, FEAT, DIM), lambda s, b: (s, 0, 0)),
        ],
        out_specs=pl.BlockSpec((1, BS, FEAT, DIM), lambda s, b: (b, s, 0, 0)),
        out_shape=jax.ShapeDtypeStruct((BATCH, SEQ, FEAT, DIM), jnp.float32),
        compiler_params=pltpu.CompilerParams(
            dimension_semantics=("arbitrary", "arbitrary"),
        ),
    )(inputs, t3)


# speedup vs baseline: 1.0049x; 1.0049x over previous
import jax
import jax.numpy as jnp
from jax.experimental import pallas as pl
from jax.experimental.pallas import tpu as pltpu

SEQ = 4096
FEAT = 26
DIM = 64
BATCH = 4

BS = 128


def _add_body(x_ref, t_ref, o_ref):
    o_ref[...] = x_ref[...] + t_ref[None]


def kernel(inputs, table):
    t3 = table.reshape(SEQ, FEAT, DIM)
    return pl.pallas_call(
        _add_body,
        grid=(SEQ // BS,),
        in_specs=[
            pl.BlockSpec((BATCH, BS, FEAT, DIM), lambda s: (0, s, 0, 0)),
            pl.BlockSpec((BS, FEAT, DIM), lambda s: (s, 0, 0)),
        ],
        out_specs=pl.BlockSpec((BATCH, BS, FEAT, DIM), lambda s: (0, s, 0, 0)),
        out_shape=jax.ShapeDtypeStruct((BATCH, SEQ, FEAT, DIM), jnp.float32),
        compiler_params=pltpu.CompilerParams(
            dimension_semantics=("arbitrary",),
        ),
    )(inputs, t3)


# speedup vs baseline: 1.0636x; 1.0584x over previous
import jax
import jax.numpy as jnp
from jax.experimental import pallas as pl
from jax.experimental.pallas import tpu as pltpu

SEQ = 4096
FEAT = 26
DIM = 64
BATCH = 4

BS = 128
NSTEP = SEQ // BS


def _x_copy(x_hbm, xb, sx, step, slot, k):
    return pltpu.make_async_copy(
        x_hbm.at[k, pl.ds(step * BS, BS)], xb.at[slot, k], sx.at[slot, k]
    )


def _t_copy(t_hbm, tb, st, step, slot):
    return pltpu.make_async_copy(
        t_hbm.at[pl.ds(step * BS, BS)], tb.at[slot], st.at[slot]
    )


def _o_copy(o_hbm, ob, so, step, slot, k):
    return pltpu.make_async_copy(
        ob.at[slot, k], o_hbm.at[k, pl.ds(step * BS, BS)], so.at[slot, k]
    )


def _body(x_hbm, t_hbm, o_hbm, xb, tb, ob, sx, st, so):
    i = pl.program_id(0)
    slot = jax.lax.rem(i, 2)
    nslot = jax.lax.rem(i + 1, 2)

    @pl.when(i == 0)
    def _():
        for k in range(BATCH):
            _x_copy(x_hbm, xb, sx, 0, 0, k).start()
        _t_copy(t_hbm, tb, st, 0, 0).start()

    @pl.when(i + 1 < NSTEP)
    def _():
        for k in range(BATCH):
            _x_copy(x_hbm, xb, sx, i + 1, nslot, k).start()
        _t_copy(t_hbm, tb, st, i + 1, nslot).start()

    for k in range(BATCH):
        _x_copy(x_hbm, xb, sx, i, slot, k).wait()
    _t_copy(t_hbm, tb, st, i, slot).wait()

    @pl.when(i >= 2)
    def _():
        for k in range(BATCH):
            _o_copy(o_hbm, ob, so, i - 2, slot, k).wait()

    t_val = tb[slot]
    for k in range(BATCH):
        ob[slot, k] = xb[slot, k] + t_val
    for k in range(BATCH):
        _o_copy(o_hbm, ob, so, i, slot, k).start()

    @pl.when(i == NSTEP - 1)
    def _():
        for k in range(BATCH):
            _o_copy(o_hbm, ob, so, i - 1, nslot, k).wait()
        for k in range(BATCH):
            _o_copy(o_hbm, ob, so, i, slot, k).wait()


def kernel(inputs, table):
    t3 = table.reshape(SEQ, FEAT, DIM)
    return pl.pallas_call(
        _body,
        grid=(NSTEP,),
        in_specs=[
            pl.BlockSpec(memory_space=pl.ANY),
            pl.BlockSpec(memory_space=pl.ANY),
        ],
        out_specs=pl.BlockSpec(memory_space=pl.ANY),
        out_shape=jax.ShapeDtypeStruct((BATCH, SEQ, FEAT, DIM), jnp.float32),
        scratch_shapes=[
            pltpu.VMEM((2, BATCH, BS, FEAT, DIM), jnp.float32),
            pltpu.VMEM((2, BS, FEAT, DIM), jnp.float32),
            pltpu.VMEM((2, BATCH, BS, FEAT, DIM), jnp.float32),
            pltpu.SemaphoreType.DMA((2, BATCH)),
            pltpu.SemaphoreType.DMA((2,)),
            pltpu.SemaphoreType.DMA((2, BATCH)),
        ],
        compiler_params=pltpu.CompilerParams(
            dimension_semantics=("arbitrary",),
        ),
    )(inputs, t3)
